# baseline (speedup 1.0000x reference)
import functools

import jax
import jax.numpy as jnp
from jax.experimental import pallas as pl
from jax.experimental.pallas import tpu as pltpu

_IGNORE_INDEX = -100
_LANES = 128
_TILE_ROWS = 2048
_CORES = 2


def _round_up(x, m):
    return ((x + m - 1) // m) * m


def _vcl_kernel(logp_ref, tgtb_ref, tgt_ref, mu_ref, lv_ref, mu_o_ref,
                lv_o_ref, out_ref, acc_ref, nll_ref, *,
                kt, tile_rows, kl_scale, stream_nll, valid_rows, needs_mask):
    c = pl.program_id(0)
    k = pl.program_id(1)

    @pl.when(k == 0)
    def _init():
        acc_ref[...] = jnp.zeros_like(acc_ref)
        nll_ref[0] = 0.0
        tgt_all = tgt_ref[...]
        valid_all = jnp.sum((tgt_all != _IGNORE_INDEX).astype(jnp.float32))
        nll_ref[1] = jnp.maximum(valid_all, 1.0)

    def nll_partial():
        logp = logp_ref[...].astype(jnp.float32)
        tgt = tgtb_ref[...]
        nb, ncls = logp.shape
        cls = jax.lax.broadcasted_iota(jnp.int32, (nb, ncls), 1)
        nll_ref[0] += jnp.sum(jnp.where(cls == tgt, logp, 0.0))

    if stream_nll:
        nll_partial()
    else:
        @pl.when(jnp.logical_and(k == 0, c == 0))
        def _nll_once():
            nll_partial()

    mu = mu_ref[...].astype(jnp.float32)
    lv = lv_ref[...].astype(jnp.float32)
    mu_o = mu_o_ref[...].astype(jnp.float32)
    lv_o = lv_o_ref[...].astype(jnp.float32)
    d = lv - lv_o
    t = (jnp.exp(d) - d) + jnp.square(mu - mu_o) * jnp.exp(-lv_o)
    if needs_mask:
        t = t - 1.0
        row0 = (c * kt + k) * tile_rows
        ridx = jax.lax.broadcasted_iota(jnp.int32, t.shape, 0)
        t = jnp.where(ridx < (valid_rows - row0), t, 0.0)
    r, w = t.shape
    acc_ref[...] += jnp.sum(t.reshape(r // 8, 8, w), axis=0)

    @pl.when(k == kt - 1)
    def _finalize():
        kl_sum = jnp.sum(acc_ref[...])
        if not needs_mask:
            kl_sum = kl_sum - float(kt * r * w)
        kl_sum = (0.5 * kl_scale) * kl_sum
        out_ref[0, 0, 0] = kl_sum - nll_ref[0] / nll_ref[1]


def kernel(output, target, mu_new, lv_new, mu_old, lv_old):
    n, ncls = output.shape
    tgt2d = target.reshape(n, 1).astype(jnp.int32)
    kl_scale = 1.0 / float(n)

    nelem = mu_new.size
    lanes = _LANES
    rows = _round_up(nelem, lanes) // lanes

    def to_rows(a):
        flat = jnp.ravel(a)
        if nelem % lanes:
            flat = jnp.pad(flat, (0, rows * lanes - nelem))
        return flat.reshape(rows, lanes)

    slabs = [to_rows(a) for a in (mu_new, lv_new, mu_old, lv_old)]

    def plan(num_cores):
        rpc = pl.cdiv(rows, num_cores)
        tr = min(_TILE_ROWS, _round_up(rpc, 8))
        return tr, pl.cdiv(rpc, tr)

    nc = _CORES
    tile_rows, kt = plan(nc)
    if nc > 1 and kt * tile_rows >= rows:
        nc = 1
        tile_rows, kt = plan(nc)

    needs_mask = (nc * kt * tile_rows != rows)
    max_block = pl.cdiv(rows, tile_rows) - 1

    def slab_map(cc, kk):
        return (jnp.minimum(cc * kt + kk, max_block), 0)

    slab_spec = pl.BlockSpec((tile_rows, lanes), slab_map)

    steps = nc * kt
    stream_nll = n % steps == 0 and (n // steps) % 8 == 0
    if stream_nll:
        nb = n // steps
        logp_spec = pl.BlockSpec((nb, ncls), lambda cc, kk: (cc * kt + kk, 0))
        tgtb_spec = pl.BlockSpec((nb, 1), lambda cc, kk: (cc * kt + kk, 0))
    else:
        logp_spec = pl.BlockSpec((n, ncls), lambda cc, kk: (0, 0))
        tgtb_spec = pl.BlockSpec((n, 1), lambda cc, kk: (0, 0))
    tgt_spec = pl.BlockSpec((n, 1), lambda cc, kk: (0, 0))

    _kernel_fn = functools.partial(
        _vcl_kernel, kt=kt, tile_rows=tile_rows, kl_scale=kl_scale,
        stream_nll=stream_nll, valid_rows=rows, needs_mask=needs_mask)

    bytes_accessed = int(sum(s.size * s.dtype.itemsize for s in slabs)
                         + output.size * output.dtype.itemsize
                         + tgt2d.size * tgt2d.dtype.itemsize + nc * 4)
    cost = pl.CostEstimate(flops=int(9 * nelem + 4 * n * ncls),
                           transcendentals=int(2 * nelem),
                           bytes_accessed=bytes_accessed)

    out = pl.pallas_call(
        _kernel_fn,
        out_shape=jax.ShapeDtypeStruct((nc, 1, 1), jnp.float32),
        grid=(nc, kt),
        in_specs=[logp_spec, tgtb_spec, tgt_spec,
                  slab_spec, slab_spec, slab_spec, slab_spec],
        out_specs=pl.BlockSpec((1, 1, 1), lambda cc, kk: (cc, 0, 0),
                               memory_space=pltpu.MemorySpace.SMEM),
        scratch_shapes=[pltpu.VMEM((8, lanes), jnp.float32),
                        pltpu.SMEM((2,), jnp.float32)],
        compiler_params=pltpu.CompilerParams(
            dimension_semantics=("parallel", "arbitrary")),
        cost_estimate=cost,
    )(output, tgt2d, tgt2d, *slabs)

    return jnp.sum(out)


# speedup vs baseline: 1.1735x; 1.1735x over previous
import functools

import jax
import jax.numpy as jnp
from jax.experimental import pallas as pl
from jax.experimental.pallas import tpu as pltpu

_IGNORE_INDEX = -100
_LANES = 128
_TILE_ROWS = 4096
_CORES = 2


def _round_up(x, m):
    return ((x + m - 1) // m) * m


def _vcl_kernel(logp_ref, tgtb_ref, tgt_ref, mu_ref, lv_ref, mu_o_ref,
                lv_o_ref, out_ref, acc_ref, nll_ref, *,
                kt, tile_rows, kl_scale, stream_nll, valid_rows, needs_mask):
    c = pl.program_id(0)
    k = pl.program_id(1)

    @pl.when(k == 0)
    def _init():
        acc_ref[...] = jnp.zeros_like(acc_ref)
        nll_ref[0] = 0.0
        tgt_all = tgt_ref[...]
        valid_all = jnp.sum((tgt_all != _IGNORE_INDEX).astype(jnp.float32))
        nll_ref[1] = jnp.maximum(valid_all, 1.0)

    def nll_partial():
        logp = logp_ref[...].astype(jnp.float32)
        tgt = tgtb_ref[...]
        nb, ncls = logp.shape
        cls = jax.lax.broadcasted_iota(jnp.int32, (nb, ncls), 1)
        nll_ref[0] += jnp.sum(jnp.where(cls == tgt, logp, 0.0))

    if stream_nll:
        nll_partial()
    else:
        @pl.when(jnp.logical_and(k == 0, c == 0))
        def _nll_once():
            nll_partial()

    mu = mu_ref[...].astype(jnp.float32)
    lv = lv_ref[...].astype(jnp.float32)
    mu_o = mu_o_ref[...].astype(jnp.float32)
    lv_o = lv_o_ref[...].astype(jnp.float32)
    d = lv - lv_o
    t = (jnp.exp(d) - d) + jnp.square(mu - mu_o) * jnp.exp(-lv_o)
    if needs_mask:
        t = t - 1.0
        row0 = (c * kt + k) * tile_rows
        ridx = jax.lax.broadcasted_iota(jnp.int32, t.shape, 0)
        t = jnp.where(ridx < (valid_rows - row0), t, 0.0)
    r, w = t.shape
    acc_ref[...] += jnp.sum(t.reshape(r // 8, 8, w), axis=0)

    @pl.when(k == kt - 1)
    def _finalize():
        kl_sum = jnp.sum(acc_ref[...])
        if not needs_mask:
            kl_sum = kl_sum - float(kt * r * w)
        kl_sum = (0.5 * kl_scale) * kl_sum
        out_ref[0, 0, 0] = kl_sum - nll_ref[0] / nll_ref[1]


def kernel(output, target, mu_new, lv_new, mu_old, lv_old):
    n, ncls = output.shape
    tgt2d = target.reshape(n, 1).astype(jnp.int32)
    kl_scale = 1.0 / float(n)

    nelem = mu_new.size
    lanes = _LANES
    rows = _round_up(nelem, lanes) // lanes

    def to_rows(a):
        flat = jnp.ravel(a)
        if nelem % lanes:
            flat = jnp.pad(flat, (0, rows * lanes - nelem))
        return flat.reshape(rows, lanes)

    slabs = [to_rows(a) for a in (mu_new, lv_new, mu_old, lv_old)]

    def plan(num_cores):
        rpc = pl.cdiv(rows, num_cores)
        tr = min(_TILE_ROWS, _round_up(rpc, 8))
        return tr, pl.cdiv(rpc, tr)

    nc = _CORES
    tile_rows, kt = plan(nc)
    if nc > 1 and kt * tile_rows >= rows:
        nc = 1
        tile_rows, kt = plan(nc)

    needs_mask = (nc * kt * tile_rows != rows)
    max_block = pl.cdiv(rows, tile_rows) - 1

    def slab_map(cc, kk):
        return (jnp.minimum(cc * kt + kk, max_block), 0)

    slab_spec = pl.BlockSpec((tile_rows, lanes), slab_map)

    steps = nc * kt
    stream_nll = n % steps == 0 and (n // steps) % 8 == 0
    if stream_nll:
        nb = n // steps
        logp_spec = pl.BlockSpec((nb, ncls), lambda cc, kk: (cc * kt + kk, 0))
        tgtb_spec = pl.BlockSpec((nb, 1), lambda cc, kk: (cc * kt + kk, 0))
    else:
        logp_spec = pl.BlockSpec((n, ncls), lambda cc, kk: (0, 0))
        tgtb_spec = pl.BlockSpec((n, 1), lambda cc, kk: (0, 0))
    tgt_spec = pl.BlockSpec((n, 1), lambda cc, kk: (0, 0))

    _kernel_fn = functools.partial(
        _vcl_kernel, kt=kt, tile_rows=tile_rows, kl_scale=kl_scale,
        stream_nll=stream_nll, valid_rows=rows, needs_mask=needs_mask)

    bytes_accessed = int(sum(s.size * s.dtype.itemsize for s in slabs)
                         + output.size * output.dtype.itemsize
                         + tgt2d.size * tgt2d.dtype.itemsize + nc * 4)
    cost = pl.CostEstimate(flops=int(9 * nelem + 4 * n * ncls),
                           transcendentals=int(2 * nelem),
                           bytes_accessed=bytes_accessed)

    out = pl.pallas_call(
        _kernel_fn,
        out_shape=jax.ShapeDtypeStruct((nc, 1, 1), jnp.float32),
        grid=(nc, kt),
        in_specs=[logp_spec, tgtb_spec, tgt_spec,
                  slab_spec, slab_spec, slab_spec, slab_spec],
        out_specs=pl.BlockSpec((1, 1, 1), lambda cc, kk: (cc, 0, 0),
                               memory_space=pltpu.MemorySpace.SMEM),
        scratch_shapes=[pltpu.VMEM((8, lanes), jnp.float32),
                        pltpu.SMEM((2,), jnp.float32)],
        compiler_params=pltpu.CompilerParams(
            dimension_semantics=("parallel", "arbitrary")),
        cost_estimate=cost,
    )(output, tgt2d, tgt2d, *slabs)

    return jnp.sum(out)
